# baseline (speedup 1.0000x reference)
import functools

import jax
import jax.numpy as jnp
from jax import lax
from jax.experimental import pallas as pl
from jax.experimental.pallas import tpu as pltpu
from jax.experimental.pallas import tpu_sc as plsc

_NMS_T = 0.3
_SCORE_T = 0.1
_K = 100
_NSUB = 16
_L = 16
_NPAD = 5120
_PER = _NPAD // _NSUB
_CHUNKS = _PER // _L
_NEG = float("-inf")
_BIGF = 1e9


def _sc_body(coords_hbm, scores_hbm, out_hbm,
             x0, y0, x1, y1, lsc, ms, areas, slotbuf, allslots, outbuf,
             shared):
    cid = lax.axis_index("c")
    sid = lax.axis_index("s")
    base = sid * _PER
    iota = lax.iota(jnp.int32, _L)
    iotaf = iota.astype(jnp.float32)

    pltpu.sync_copy(coords_hbm.at[0], x0)
    pltpu.sync_copy(coords_hbm.at[1], y0)
    pltpu.sync_copy(coords_hbm.at[2], x1)
    pltpu.sync_copy(coords_hbm.at[3], y1)
    pltpu.sync_copy(scores_hbm.at[pl.ds(base, _PER)], lsc)

    for j in range(_CHUNKS):
        sl = pl.ds(j * _L, _L)
        dsl = pl.ds(base + j * _L, _L)
        a = x0[dsl]
        b = y0[dsl]
        c = x1[dsl]
        d = y1[dsl]
        sc = lsc[sl]
        valid = (c > a) & (d > b) & (sc > _SCORE_T)
        ms[sl] = jnp.where(valid, sc, _NEG)
        areas[sl] = jnp.maximum(c - a, 0.0) * jnp.maximum(d - b, 0.0)

    def round_fn(k, carry):
        mvec = ms[pl.ds(0, _L)]
        for j in range(1, _CHUNKS):
            mvec = jnp.maximum(mvec, ms[pl.ds(j * _L, _L)])
        m_loc = jnp.max(mvec)
        ivec = jnp.full((_L,), _BIGF, jnp.float32)
        for j in range(_CHUNKS):
            gidx = iotaf + jnp.float32(base + j * _L)
            cand = jnp.where(ms[pl.ds(j * _L, _L)] == m_loc, gidx, _BIGF)
            ivec = jnp.minimum(ivec, cand)
        i_loc = jnp.min(ivec)

        slot = jnp.where(iota == 0, m_loc, jnp.where(iota == 1, i_loc, 0.0))
        slotbuf[...] = slot
        pltpu.sync_copy(slotbuf, shared.at[pl.ds(sid * _L, _L)])
        plsc.subcore_barrier()
        pltpu.sync_copy(shared, allslots)
        plsc.subcore_barrier()
        vals = plsc.load_gather(allslots, [iota * _L])
        idxs = plsc.load_gather(allslots, [iota * _L + 1])
        m = jnp.max(vals)
        wi_f = jnp.min(jnp.where(vals == m, idxs, _BIGF))
        finite = m > _NEG
        wii = jnp.where(finite, wi_f, 0.0).astype(jnp.int32)
        widx = jnp.broadcast_to(wii, (_L,))
        wx0 = jnp.where(finite, plsc.load_gather(x0, [widx]), 0.0)
        wy0 = jnp.where(finite, plsc.load_gather(y0, [widx]), 0.0)
        wx1 = jnp.where(finite, plsc.load_gather(x1, [widx]), 0.0)
        wy1 = jnp.where(finite, plsc.load_gather(y1, [widx]), 0.0)
        w_area = (jnp.maximum(wx1 - wx0, 0.0) * jnp.maximum(wy1 - wy0, 0.0))

        for j in range(_CHUNKS):
            sl = pl.ds(j * _L, _L)
            dsl = pl.ds(base + j * _L, _L)
            a = x0[dsl]
            b = y0[dsl]
            c = x1[dsl]
            d = y1[dsl]
            ix0 = jnp.maximum(a, wx0)
            iy0 = jnp.maximum(b, wy0)
            ix1 = jnp.minimum(c, wx1)
            iy1 = jnp.minimum(d, wy1)
            inter = (jnp.maximum(ix1 - ix0, 0.0)
                     * jnp.maximum(iy1 - iy0, 0.0))
            union = areas[sl] + w_area - inter
            iou = inter / jnp.maximum(union, 1e-9)
            ms[sl] = jnp.where(iou > _NMS_T, _NEG, ms[sl])

        @pl.when(jnp.logical_and(cid == 0, sid == 0))
        def _():
            m_out = jnp.where(finite, m, 0.0)
            row = jnp.where(iota == 0, wx0,
                  jnp.where(iota == 1, wy0,
                  jnp.where(iota == 2, wx1,
                  jnp.where(iota == 3, wy1,
                  jnp.where(iota == 4, jnp.broadcast_to(m_out, (_L,)),
                            0.0)))))
            plsc.store_scatter(outbuf, [k * _L + iota], row)

        return carry

    lax.fori_loop(0, _K, round_fn, 0)

    @pl.when(jnp.logical_and(cid == 0, sid == 0))
    def _():
        pltpu.sync_copy(outbuf, out_hbm)


@jax.jit
def kernel(boxes, scores):
    n = boxes.shape[0]
    pad = _NPAD - n
    coords = jnp.pad(boxes.T, ((0, 0), (0, pad)))
    sp = jnp.pad(scores, (0, pad))
    mesh = plsc.VectorSubcoreMesh(core_axis_name="c", subcore_axis_name="s",
                                  num_cores=2, num_subcores=_NSUB)
    f = pl.kernel(
        _sc_body,
        out_type=jax.ShapeDtypeStruct((_K * _L,), jnp.float32),
        mesh=mesh,
        compiler_params=pltpu.CompilerParams(needs_layout_passes=False),
        scratch_types=[
            pltpu.VMEM((_NPAD,), jnp.float32),
            pltpu.VMEM((_NPAD,), jnp.float32),
            pltpu.VMEM((_NPAD,), jnp.float32),
            pltpu.VMEM((_NPAD,), jnp.float32),
            pltpu.VMEM((_PER,), jnp.float32),
            pltpu.VMEM((_PER,), jnp.float32),
            pltpu.VMEM((_PER,), jnp.float32),
            pltpu.VMEM((_L,), jnp.float32),
            pltpu.VMEM((_NSUB * _L,), jnp.float32),
            pltpu.VMEM((_K * _L,), jnp.float32),
            pltpu.VMEM_SHARED((_NSUB * _L,), jnp.float32),
        ],
    )
    out = f(coords, sp)
    return out.reshape(_K, _L)[:, :5]


# speedup vs baseline: 1.5518x; 1.5518x over previous
import functools

import jax
import jax.numpy as jnp
from jax import lax
from jax.experimental import pallas as pl
from jax.experimental.pallas import tpu as pltpu
from jax.experimental.pallas import tpu_sc as plsc

_NMS_T = 0.3
_SCORE_T = 0.1
_K = 100
_NSUB = 16
_L = 16
_NPAD = 5120
_PER = _NPAD // _NSUB
_CHUNKS = _PER // _L
_SLOTS = _NSUB * _L
_NEG = float("-inf")
_BIGF = 1e9


def _sc_body(coords_hbm, scores_hbm, out_hbm,
             x0, y0, x1, y1, lx0, ly0, lx1, ly1, ms, areas,
             slotbuf, allslots, outbuf, shared):
    cid = lax.axis_index("c")
    sid = lax.axis_index("s")
    base = sid * _PER
    iota = lax.iota(jnp.int32, _L)
    iotaf = iota.astype(jnp.float32)

    pltpu.sync_copy(coords_hbm.at[0], x0)
    pltpu.sync_copy(coords_hbm.at[1], y0)
    pltpu.sync_copy(coords_hbm.at[2], x1)
    pltpu.sync_copy(coords_hbm.at[3], y1)
    pltpu.sync_copy(scores_hbm.at[pl.ds(base, _PER)], ms)

    for j in range(_CHUNKS):
        sl = pl.ds(j * _L, _L)
        dsl = pl.ds(base + j * _L, _L)
        a = x0[dsl]
        b = y0[dsl]
        c = x1[dsl]
        d = y1[dsl]
        lx0[sl] = a
        ly0[sl] = b
        lx1[sl] = c
        ly1[sl] = d
        sc = ms[sl]
        valid = (c > a) & (d > b) & (sc > _SCORE_T)
        ms[sl] = jnp.where(valid, sc, _NEG)
        areas[sl] = jnp.maximum(c - a, 0.0) * jnp.maximum(d - b, 0.0)

    basef = base.astype(jnp.float32)
    gidx0 = iotaf + basef

    def round_fn(k, carry):
        wx0, wy0, wx1, wy1, w_area = carry
        m1 = jnp.full((_L,), _NEG, jnp.float32)
        iv = jnp.full((_L,), _BIGF, jnp.float32)
        for j in range(_CHUNKS):
            sl = pl.ds(j * _L, _L)
            a = lx0[sl]
            b = ly0[sl]
            c = lx1[sl]
            d = ly1[sl]
            ix0 = jnp.maximum(a, wx0)
            iy0 = jnp.maximum(b, wy0)
            ix1 = jnp.minimum(c, wx1)
            iy1 = jnp.minimum(d, wy1)
            inter = (jnp.maximum(ix1 - ix0, 0.0)
                     * jnp.maximum(iy1 - iy0, 0.0))
            union = areas[sl] + w_area - inter
            iou = inter / jnp.maximum(union, 1e-9)
            msn = jnp.where(iou > _NMS_T, _NEG, ms[sl])
            ms[sl] = msn
            upd = msn > m1
            m1 = jnp.where(upd, msn, m1)
            iv = jnp.where(upd, gidx0 + jnp.float32(j * _L), iv)
        m_loc = jnp.max(m1)
        i_loc = jnp.min(jnp.where(m1 == m_loc, iv, _BIGF))

        slot = jnp.where(iota == 0, m_loc, jnp.where(iota == 1, i_loc, 0.0))
        slotbuf[...] = slot
        par = (k & 1) * _SLOTS
        pltpu.sync_copy(slotbuf, shared.at[pl.ds(par + sid * _L, _L)])
        plsc.subcore_barrier()
        pltpu.sync_copy(shared.at[pl.ds(par, _SLOTS)], allslots)
        vals = plsc.load_gather(allslots, [iota * _L])
        idxs = plsc.load_gather(allslots, [iota * _L + 1])
        m = jnp.max(vals)
        wi_f = jnp.min(jnp.where(vals == m, idxs, _BIGF))
        finite = m > _NEG
        wii = jnp.where(finite, wi_f, 0.0).astype(jnp.int32)
        widx = jnp.broadcast_to(wii, (_L,))
        nwx0 = jnp.where(finite, plsc.load_gather(x0, [widx]), 0.0)
        nwy0 = jnp.where(finite, plsc.load_gather(y0, [widx]), 0.0)
        nwx1 = jnp.where(finite, plsc.load_gather(x1, [widx]), 0.0)
        nwy1 = jnp.where(finite, plsc.load_gather(y1, [widx]), 0.0)
        nw_area = (jnp.maximum(nwx1 - nwx0, 0.0)
                   * jnp.maximum(nwy1 - nwy0, 0.0))

        @pl.when(jnp.logical_and(cid == 0, sid == 0))
        def _():
            m_out = jnp.where(finite, m, 0.0)
            row = jnp.where(iota == 0, nwx0,
                  jnp.where(iota == 1, nwy0,
                  jnp.where(iota == 2, nwx1,
                  jnp.where(iota == 3, nwy1,
                  jnp.where(iota == 4, jnp.broadcast_to(m_out, (_L,)),
                            0.0)))))
            plsc.store_scatter(outbuf, [k * _L + iota], row)

        return (nwx0, nwy0, nwx1, nwy1, nw_area)

    zero = jnp.zeros((_L,), jnp.float32)
    lax.fori_loop(0, _K, round_fn, (zero, zero, zero, zero, zero))

    @pl.when(jnp.logical_and(cid == 0, sid == 0))
    def _():
        pltpu.sync_copy(outbuf, out_hbm)


@jax.jit
def kernel(boxes, scores):
    n = boxes.shape[0]
    pad = _NPAD - n
    coords = jnp.pad(boxes.T, ((0, 0), (0, pad)))
    sp = jnp.pad(scores, (0, pad))
    mesh = plsc.VectorSubcoreMesh(core_axis_name="c", subcore_axis_name="s",
                                  num_cores=2, num_subcores=_NSUB)
    f = pl.kernel(
        _sc_body,
        out_type=jax.ShapeDtypeStruct((_K * _L,), jnp.float32),
        mesh=mesh,
        compiler_params=pltpu.CompilerParams(needs_layout_passes=False),
        scratch_types=[
            pltpu.VMEM((_NPAD,), jnp.float32),
            pltpu.VMEM((_NPAD,), jnp.float32),
            pltpu.VMEM((_NPAD,), jnp.float32),
            pltpu.VMEM((_NPAD,), jnp.float32),
            pltpu.VMEM((_PER,), jnp.float32),
            pltpu.VMEM((_PER,), jnp.float32),
            pltpu.VMEM((_PER,), jnp.float32),
            pltpu.VMEM((_PER,), jnp.float32),
            pltpu.VMEM((_PER,), jnp.float32),
            pltpu.VMEM((_PER,), jnp.float32),
            pltpu.VMEM((_L,), jnp.float32),
            pltpu.VMEM((_SLOTS,), jnp.float32),
            pltpu.VMEM((_K * _L,), jnp.float32),
            pltpu.VMEM_SHARED((2 * _SLOTS,), jnp.float32),
        ],
    )
    out = f(coords, sp)
    return out.reshape(_K, _L)[:, :5]


# speedup vs baseline: 1.8681x; 1.2039x over previous
import functools

import jax
import jax.numpy as jnp
from jax import lax
from jax.experimental import pallas as pl
from jax.experimental.pallas import tpu as pltpu
from jax.experimental.pallas import tpu_sc as plsc

_NMS_T = 0.3
_SCORE_T = 0.1
_K = 100
_NSUB = 16
_L = 16
_NPAD = 5120
_PER = _NPAD // _NSUB
_CHUNKS = _PER // _L
_SLOTS = _NSUB * _L
_B = 4
_NEG = float("-inf")
_BIGF = 1e9


def _iou_vs(wx0, wy0, wx1, wy1, wa, a, b, c, d, area):
    ix0 = jnp.maximum(a, wx0)
    iy0 = jnp.maximum(b, wy0)
    ix1 = jnp.minimum(c, wx1)
    iy1 = jnp.minimum(d, wy1)
    inter = jnp.maximum(ix1 - ix0, 0.0) * jnp.maximum(iy1 - iy0, 0.0)
    union = area + wa - inter
    return inter / jnp.maximum(union, 1e-9)


def _sc_body(coords_hbm, scores_hbm, out_hbm,
             x0, y0, x1, y1, lx0, ly0, lx1, ly1, ms,
             slotbuf, allslots, outbuf, shared):
    cid = lax.axis_index("c")
    sid = lax.axis_index("s")
    base = sid * _PER
    iota = lax.iota(jnp.int32, _L)
    iotaf = iota.astype(jnp.float32)
    is_out = jnp.logical_and(cid == 0, sid == 0)

    pltpu.sync_copy(coords_hbm.at[0], x0)
    pltpu.sync_copy(coords_hbm.at[1], y0)
    pltpu.sync_copy(coords_hbm.at[2], x1)
    pltpu.sync_copy(coords_hbm.at[3], y1)
    pltpu.sync_copy(scores_hbm.at[pl.ds(base, _PER)], ms)

    for j in range(_CHUNKS):
        sl = pl.ds(j * _L, _L)
        dsl = pl.ds(base + j * _L, _L)
        a = x0[dsl]
        b = y0[dsl]
        c = x1[dsl]
        d = y1[dsl]
        lx0[sl] = a
        ly0[sl] = b
        lx1[sl] = c
        ly1[sl] = d
        sc = ms[sl]
        valid = (c > a) & (d > b) & (sc > _SCORE_T)
        ms[sl] = jnp.where(valid, sc, _NEG)

    basef = base.astype(jnp.float32)
    gidx0 = iotaf + basef
    zero = jnp.zeros((_L,), jnp.float32)

    def cond_fn(carry):
        rnd, kc, cont = carry[0], carry[1], carry[2]
        return jnp.logical_and(kc < _K, cont > 0)

    def round_fn(carry):
        rnd, kc = carry[0], carry[1]
        wins = carry[3:]
        m1 = jnp.full((_L,), _NEG, jnp.float32)
        m2 = jnp.full((_L,), _NEG, jnp.float32)
        iv = jnp.full((_L,), _BIGF, jnp.float32)
        for j in range(_CHUNKS):
            sl = pl.ds(j * _L, _L)
            a = lx0[sl]
            b = ly0[sl]
            c = lx1[sl]
            d = ly1[sl]
            area = (c - a) * (d - b)
            msn = ms[sl]
            for t in range(_B):
                wx0, wy0, wx1, wy1, wa = wins[5 * t:5 * t + 5]
                iou = _iou_vs(wx0, wy0, wx1, wy1, wa, a, b, c, d, area)
                msn = jnp.where(iou > _NMS_T, _NEG, msn)
            ms[sl] = msn
            upd = msn > m1
            m2 = jnp.maximum(m2, jnp.minimum(m1, msn))
            m1 = jnp.where(upd, msn, m1)
            iv = jnp.where(upd, gidx0 + jnp.float32(j * _L), iv)
        m_loc = jnp.max(m1)
        eq = m1 == m_loc
        i_loc = jnp.min(jnp.where(eq, iv, _BIGF))
        ncnt = plsc.all_reduce_population_count(eq)
        e1 = jnp.max(jnp.where(eq, _NEG, m1))
        e2 = jnp.max(jnp.where(eq, m2, _NEG))
        m2_loc = jnp.where(ncnt >= 2, m_loc, jnp.maximum(e1, e2))

        slot = jnp.where(iota == 0, m_loc,
               jnp.where(iota == 1, i_loc,
               jnp.where(iota == 2, m2_loc, 0.0)))
        slotbuf[...] = slot
        par = (rnd & 1) * _SLOTS
        pltpu.sync_copy(slotbuf, shared.at[pl.ds(par + sid * _L, _L)])
        plsc.subcore_barrier()
        pltpu.sync_copy(shared.at[pl.ds(par, _SLOTS)], allslots)
        vals = plsc.load_gather(allslots, [iota * _L])
        idxs = plsc.load_gather(allslots, [iota * _L + 1])
        m2s = plsc.load_gather(allslots, [iota * _L + 2])
        bound = jnp.max(m2s)

        pci = jnp.where(vals > _NEG, idxs, 0.0).astype(jnp.int32)
        px0 = plsc.load_gather(x0, [pci])
        py0 = plsc.load_gather(y0, [pci])
        px1 = plsc.load_gather(x1, [pci])
        py1 = plsc.load_gather(y1, [pci])
        parea = (px1 - px0) * (py1 - py0)

        pv = vals
        ka = jnp.int32(0)
        new_wins = []
        for t in range(_B):
            m_t = jnp.max(pv)
            wi = jnp.min(jnp.where(pv == m_t, idxs, _BIGF))
            ok = m_t > _NEG
            if t == 0:
                acc = jnp.logical_and(ok, kc + ka < _K)
            else:
                acc = jnp.logical_and(jnp.logical_and(ok, m_t > bound),
                                      kc + ka < _K)
            wii = jnp.where(acc, wi, 0.0).astype(jnp.int32)
            widx = jnp.broadcast_to(wii, (_L,))
            nwx0 = jnp.where(acc, plsc.load_gather(x0, [widx]), 0.0)
            nwy0 = jnp.where(acc, plsc.load_gather(y0, [widx]), 0.0)
            nwx1 = jnp.where(acc, plsc.load_gather(x1, [widx]), 0.0)
            nwy1 = jnp.where(acc, plsc.load_gather(y1, [widx]), 0.0)
            nwa = (jnp.maximum(nwx1 - nwx0, 0.0)
                   * jnp.maximum(nwy1 - nwy0, 0.0))
            new_wins += [nwx0, nwy0, nwx1, nwy1, nwa]

            @pl.when(jnp.logical_and(acc, is_out))
            def _():
                row = jnp.where(iota == 0, nwx0,
                      jnp.where(iota == 1, nwy0,
                      jnp.where(iota == 2, nwx1,
                      jnp.where(iota == 3, nwy1,
                      jnp.where(iota == 4, jnp.broadcast_to(m_t, (_L,)),
                                0.0)))))
                plsc.store_scatter(outbuf, [(kc + ka) * _L + iota], row)

            piou = _iou_vs(nwx0, nwy0, nwx1, nwy1, nwa,
                           px0, py0, px1, py1, parea)
            pv = jnp.where(jnp.logical_and(acc, piou > _NMS_T), _NEG, pv)
            ka = ka + acc.astype(jnp.int32)

        return tuple([rnd + 1, kc + ka, ka] + new_wins)

    init = tuple([jnp.int32(0), jnp.int32(0), jnp.int32(1)]
                 + [zero] * (5 * _B))
    final = lax.while_loop(cond_fn, round_fn, init)
    kc_end = final[1]

    @pl.when(is_out)
    def _():
        def fill(r, c):
            plsc.store_scatter(outbuf, [r * _L + iota], zero)
            return c
        lax.fori_loop(kc_end, _K, fill, 0)
        pltpu.sync_copy(outbuf, out_hbm)


@jax.jit
def kernel(boxes, scores):
    n = boxes.shape[0]
    pad = _NPAD - n
    coords = jnp.pad(boxes.T, ((0, 0), (0, pad)))
    sp = jnp.pad(scores, (0, pad))
    mesh = plsc.VectorSubcoreMesh(core_axis_name="c", subcore_axis_name="s",
                                  num_cores=2, num_subcores=_NSUB)
    f = pl.kernel(
        _sc_body,
        out_type=jax.ShapeDtypeStruct((_K * _L,), jnp.float32),
        mesh=mesh,
        compiler_params=pltpu.CompilerParams(needs_layout_passes=False),
        scratch_types=[
            pltpu.VMEM((_NPAD,), jnp.float32),
            pltpu.VMEM((_NPAD,), jnp.float32),
            pltpu.VMEM((_NPAD,), jnp.float32),
            pltpu.VMEM((_NPAD,), jnp.float32),
            pltpu.VMEM((_PER,), jnp.float32),
            pltpu.VMEM((_PER,), jnp.float32),
            pltpu.VMEM((_PER,), jnp.float32),
            pltpu.VMEM((_PER,), jnp.float32),
            pltpu.VMEM((_PER,), jnp.float32),
            pltpu.VMEM((_L,), jnp.float32),
            pltpu.VMEM((_SLOTS,), jnp.float32),
            pltpu.VMEM((_K * _L,), jnp.float32),
            pltpu.VMEM_SHARED((2 * _SLOTS,), jnp.float32),
        ],
    )
    out = f(coords, sp)
    return out.reshape(_K, _L)[:, :5]


# speedup vs baseline: 1.9401x; 1.0385x over previous
import functools

import jax
import jax.numpy as jnp
from jax import lax
from jax.experimental import pallas as pl
from jax.experimental.pallas import tpu as pltpu
from jax.experimental.pallas import tpu_sc as plsc

_NMS_T = 0.3
_SCORE_T = 0.1
_K = 100
_NSUB = 16
_L = 16
_NPAD = 5120
_PER = _NPAD // _NSUB
_CHUNKS = _PER // _L
_SLOTS = _NSUB * _L
_B = 4
_NEG = float("-inf")
_BIGF = 1e9


def _sup_vs(wx0, wy0, wx1, wy1, wa, a, b, c, d, area):
    ix0 = jnp.maximum(a, wx0)
    iy0 = jnp.maximum(b, wy0)
    ix1 = jnp.minimum(c, wx1)
    iy1 = jnp.minimum(d, wy1)
    inter = jnp.maximum(ix1 - ix0, 0.0) * jnp.maximum(iy1 - iy0, 0.0)
    union = area + wa - inter
    return inter > _NMS_T * union


def _sc_body(coords_hbm, scores_hbm, out_hbm,
             x0, y0, x1, y1, lx0, ly0, lx1, ly1, ms,
             slotbuf, allslots, outbuf, shared):
    cid = lax.axis_index("c")
    sid = lax.axis_index("s")
    base = sid * _PER
    iota = lax.iota(jnp.int32, _L)
    iotaf = iota.astype(jnp.float32)
    is_out = jnp.logical_and(cid == 0, sid == 0)

    pltpu.sync_copy(coords_hbm.at[0], x0)
    pltpu.sync_copy(coords_hbm.at[1], y0)
    pltpu.sync_copy(coords_hbm.at[2], x1)
    pltpu.sync_copy(coords_hbm.at[3], y1)
    pltpu.sync_copy(scores_hbm.at[pl.ds(base, _PER)], ms)

    for j in range(_CHUNKS):
        sl = pl.ds(j * _L, _L)
        dsl = pl.ds(base + j * _L, _L)
        a = x0[dsl]
        b = y0[dsl]
        c = x1[dsl]
        d = y1[dsl]
        lx0[sl] = a
        ly0[sl] = b
        lx1[sl] = c
        ly1[sl] = d
        sc = ms[sl]
        valid = (c > a) & (d > b) & (sc > _SCORE_T)
        ms[sl] = jnp.where(valid, sc, _NEG)

    basef = base.astype(jnp.float32)
    gidx0 = iotaf + basef
    zero = jnp.zeros((_L,), jnp.float32)

    def cond_fn(carry):
        rnd, kc, cont = carry[0], carry[1], carry[2]
        return jnp.logical_and(kc < _K, cont > 0)

    def round_fn(carry):
        rnd, kc = carry[0], carry[1]
        wins = carry[3:]
        m1 = jnp.full((_L,), _NEG, jnp.float32)
        m2 = jnp.full((_L,), _NEG, jnp.float32)
        iv = jnp.full((_L,), _BIGF, jnp.float32)
        for j in range(_CHUNKS):
            sl = pl.ds(j * _L, _L)
            a = lx0[sl]
            b = ly0[sl]
            c = lx1[sl]
            d = ly1[sl]
            area = (c - a) * (d - b)
            msn = ms[sl]
            for t in range(_B):
                wx0, wy0, wx1, wy1, wa = wins[5 * t:5 * t + 5]
                sup = _sup_vs(wx0, wy0, wx1, wy1, wa, a, b, c, d, area)
                msn = jnp.where(sup, _NEG, msn)
            ms[sl] = msn
            upd = msn > m1
            m2 = jnp.maximum(m2, jnp.minimum(m1, msn))
            m1 = jnp.where(upd, msn, m1)
            iv = jnp.where(upd, gidx0 + jnp.float32(j * _L), iv)
        m_loc = jnp.max(m1)
        eq = m1 == m_loc
        i_loc = jnp.min(jnp.where(eq, iv, _BIGF))
        ncnt = plsc.all_reduce_population_count(eq)
        e1 = jnp.max(jnp.where(eq, _NEG, m1))
        e2 = jnp.max(jnp.where(eq, m2, _NEG))
        m2_loc = jnp.where(ncnt >= 2, m_loc, jnp.maximum(e1, e2))

        slot = jnp.where(iota == 0, m_loc,
               jnp.where(iota == 1, i_loc,
               jnp.where(iota == 2, m2_loc, 0.0)))
        slotbuf[...] = slot
        par = (rnd & 1) * _SLOTS
        pltpu.sync_copy(slotbuf, shared.at[pl.ds(par + sid * _L, _L)])
        plsc.subcore_barrier()
        pltpu.sync_copy(shared.at[pl.ds(par, _SLOTS)], allslots)
        vals = plsc.load_gather(allslots, [iota * _L])
        idxs = plsc.load_gather(allslots, [iota * _L + 1])
        m2s = plsc.load_gather(allslots, [iota * _L + 2])
        bound = jnp.max(m2s)

        pci = jnp.where(vals > _NEG, idxs, 0.0).astype(jnp.int32)
        px0 = plsc.load_gather(x0, [pci])
        py0 = plsc.load_gather(y0, [pci])
        px1 = plsc.load_gather(x1, [pci])
        py1 = plsc.load_gather(y1, [pci])
        parea = (px1 - px0) * (py1 - py0)

        pv = vals
        ka = jnp.int32(0)
        new_wins = []
        for t in range(_B):
            m_t = jnp.max(pv)
            wi = jnp.min(jnp.where(pv == m_t, idxs, _BIGF))
            ok = m_t > _NEG
            if t == 0:
                acc = jnp.logical_and(ok, kc + ka < _K)
            else:
                acc = jnp.logical_and(jnp.logical_and(ok, m_t > bound),
                                      kc + ka < _K)
            wii = jnp.where(acc, wi, 0.0).astype(jnp.int32)
            widx = jnp.broadcast_to(wii, (_L,))
            nwx0 = jnp.where(acc, plsc.load_gather(x0, [widx]), 0.0)
            nwy0 = jnp.where(acc, plsc.load_gather(y0, [widx]), 0.0)
            nwx1 = jnp.where(acc, plsc.load_gather(x1, [widx]), 0.0)
            nwy1 = jnp.where(acc, plsc.load_gather(y1, [widx]), 0.0)
            nwa = (jnp.maximum(nwx1 - nwx0, 0.0)
                   * jnp.maximum(nwy1 - nwy0, 0.0))
            new_wins += [nwx0, nwy0, nwx1, nwy1, nwa]

            @pl.when(jnp.logical_and(acc, is_out))
            def _():
                row = jnp.where(iota == 0, nwx0,
                      jnp.where(iota == 1, nwy0,
                      jnp.where(iota == 2, nwx1,
                      jnp.where(iota == 3, nwy1,
                      jnp.where(iota == 4, jnp.broadcast_to(m_t, (_L,)),
                                0.0)))))
                plsc.store_scatter(outbuf, [(kc + ka) * _L + iota], row)

            psup = _sup_vs(nwx0, nwy0, nwx1, nwy1, nwa,
                           px0, py0, px1, py1, parea)
            pv = jnp.where(jnp.logical_and(acc, psup), _NEG, pv)
            ka = ka + acc.astype(jnp.int32)

        return tuple([rnd + 1, kc + ka, ka] + new_wins)

    init = tuple([jnp.int32(0), jnp.int32(0), jnp.int32(1)]
                 + [zero] * (5 * _B))
    final = lax.while_loop(cond_fn, round_fn, init)
    kc_end = final[1]

    @pl.when(is_out)
    def _():
        def fill(r, c):
            plsc.store_scatter(outbuf, [r * _L + iota], zero)
            return c
        lax.fori_loop(kc_end, _K, fill, 0)
        pltpu.sync_copy(outbuf, out_hbm)


@jax.jit
def kernel(boxes, scores):
    n = boxes.shape[0]
    pad = _NPAD - n
    coords = jnp.pad(boxes.T, ((0, 0), (0, pad)))
    sp = jnp.pad(scores, (0, pad))
    mesh = plsc.VectorSubcoreMesh(core_axis_name="c", subcore_axis_name="s",
                                  num_cores=2, num_subcores=_NSUB)
    f = pl.kernel(
        _sc_body,
        out_type=jax.ShapeDtypeStruct((_K * _L,), jnp.float32),
        mesh=mesh,
        compiler_params=pltpu.CompilerParams(needs_layout_passes=False),
        scratch_types=[
            pltpu.VMEM((_NPAD,), jnp.float32),
            pltpu.VMEM((_NPAD,), jnp.float32),
            pltpu.VMEM((_NPAD,), jnp.float32),
            pltpu.VMEM((_NPAD,), jnp.float32),
            pltpu.VMEM((_PER,), jnp.float32),
            pltpu.VMEM((_PER,), jnp.float32),
            pltpu.VMEM((_PER,), jnp.float32),
            pltpu.VMEM((_PER,), jnp.float32),
            pltpu.VMEM((_PER,), jnp.float32),
            pltpu.VMEM((_L,), jnp.float32),
            pltpu.VMEM((_SLOTS,), jnp.float32),
            pltpu.VMEM((_K * _L,), jnp.float32),
            pltpu.VMEM_SHARED((2 * _SLOTS,), jnp.float32),
        ],
    )
    out = f(coords, sp)
    return out.reshape(_K, _L)[:, :5]


# speedup vs baseline: 1.9740x; 1.0174x over previous
import functools

import jax
import jax.numpy as jnp
from jax import lax
from jax.experimental import pallas as pl
from jax.experimental.pallas import tpu as pltpu
from jax.experimental.pallas import tpu_sc as plsc

_NMS_T = 0.3
_SCORE_T = 0.1
_K = 100
_NSUB = 16
_L = 16
_N = 5000
_NPAD = 5120
_PER = _NPAD // _NSUB
_CHUNKS = _PER // _L
_SLOTS = _NSUB * _L
_B = 4
_NEG = float("-inf")
_BIGF = 1e9


def _sup_vs(wx0, wy0, wx1, wy1, wa, a, b, c, d, area):
    ix0 = jnp.maximum(a, wx0)
    iy0 = jnp.maximum(b, wy0)
    ix1 = jnp.minimum(c, wx1)
    iy1 = jnp.minimum(d, wy1)
    inter = jnp.maximum(ix1 - ix0, 0.0) * jnp.maximum(iy1 - iy0, 0.0)
    union = area + wa - inter
    return inter > _NMS_T * union


def _sc_body(boxes_hbm, scores_hbm, out_hbm,
             bx, sv, lx0, ly0, lx1, ly1, ms,
             slotbuf, allslots, outbuf, shared):
    cid = lax.axis_index("c")
    sid = lax.axis_index("s")
    base = sid * _PER
    iota = lax.iota(jnp.int32, _L)
    iotaf = iota.astype(jnp.float32)
    is_out = jnp.logical_and(cid == 0, sid == 0)

    zero = jnp.zeros((_L,), jnp.float32)
    for j in range(4 * _N // _L, 4 * _NPAD // _L):
        bx[pl.ds(j * _L, _L)] = zero
    for j in range(_N // _L, _NPAD // _L):
        sv[pl.ds(j * _L, _L)] = zero
    pltpu.sync_copy(boxes_hbm, bx.at[pl.ds(0, 4 * _N)])
    pltpu.sync_copy(scores_hbm, sv.at[pl.ds(0, _N)])

    for j in range(_CHUNKS):
        sl = pl.ds(j * _L, _L)
        gi4 = (base + j * _L + iota) * 4
        a = plsc.load_gather(bx, [gi4])
        b = plsc.load_gather(bx, [gi4 + 1])
        c = plsc.load_gather(bx, [gi4 + 2])
        d = plsc.load_gather(bx, [gi4 + 3])
        lx0[sl] = a
        ly0[sl] = b
        lx1[sl] = c
        ly1[sl] = d
        sc = sv[pl.ds(base + j * _L, _L)]
        valid = (c > a) & (d > b) & (sc > _SCORE_T)
        ms[sl] = jnp.where(valid, sc, _NEG)

    basef = base.astype(jnp.float32)
    gidx0 = iotaf + basef

    def cond_fn(carry):
        return jnp.logical_and(carry[1] < _K, carry[2] > 0)

    def round_fn(carry):
        rnd, kc = carry[0], carry[1]
        wins = carry[3:]
        m1 = jnp.full((_L,), _NEG, jnp.float32)
        m2 = jnp.full((_L,), _NEG, jnp.float32)
        iv = jnp.full((_L,), _BIGF, jnp.float32)
        for j in range(_CHUNKS):
            sl = pl.ds(j * _L, _L)
            a = lx0[sl]
            b = ly0[sl]
            c = lx1[sl]
            d = ly1[sl]
            area = (c - a) * (d - b)
            msn = ms[sl]
            for t in range(_B):
                wx0, wy0, wx1, wy1, wa = wins[5 * t:5 * t + 5]
                sup = _sup_vs(wx0, wy0, wx1, wy1, wa, a, b, c, d, area)
                msn = jnp.where(sup, _NEG, msn)
            ms[sl] = msn
            upd = msn > m1
            m2 = jnp.maximum(m2, jnp.minimum(m1, msn))
            m1 = jnp.where(upd, msn, m1)
            iv = jnp.where(upd, gidx0 + jnp.float32(j * _L), iv)
        m_loc = jnp.max(m1)
        eq = m1 == m_loc
        i_loc = jnp.min(jnp.where(eq, iv, _BIGF))
        ncnt = plsc.all_reduce_population_count(eq)
        zvec = jnp.where(eq, m2, m1)
        m2_loc = jnp.where(ncnt >= 2, m_loc, jnp.max(zvec))

        slot = jnp.where(iota == 0, m_loc,
               jnp.where(iota == 1, i_loc,
               jnp.where(iota == 2, m2_loc, 0.0)))
        slotbuf[...] = slot
        par = (rnd & 1) * _SLOTS
        pltpu.sync_copy(slotbuf, shared.at[pl.ds(par + sid * _L, _L)])
        plsc.subcore_barrier()
        pltpu.sync_copy(shared.at[pl.ds(par, _SLOTS)], allslots)
        vals = plsc.load_gather(allslots, [iota * _L])
        idxs = plsc.load_gather(allslots, [iota * _L + 1])
        m2s = plsc.load_gather(allslots, [iota * _L + 2])
        bound = jnp.max(m2s)

        pci = jnp.where(vals > _NEG, idxs, 0.0).astype(jnp.int32) * 4
        px0 = plsc.load_gather(bx, [pci])
        py0 = plsc.load_gather(bx, [pci + 1])
        px1 = plsc.load_gather(bx, [pci + 2])
        py1 = plsc.load_gather(bx, [pci + 3])
        parea = (px1 - px0) * (py1 - py0)

        pv = vals
        ka = jnp.int32(0)
        new_wins = []
        for t in range(_B):
            m_t = jnp.max(pv)
            wi = jnp.min(jnp.where(pv == m_t, idxs, _BIGF))
            ok = m_t > _NEG
            if t == 0:
                acc = jnp.logical_and(ok, kc + ka < _K)
            else:
                acc = jnp.logical_and(jnp.logical_and(ok, m_t > bound),
                                      kc + ka < _K)
            wii = jnp.where(acc, wi, 0.0).astype(jnp.int32) * 4
            widx = jnp.broadcast_to(wii, (_L,))
            nwx0 = jnp.where(acc, plsc.load_gather(bx, [widx]), 0.0)
            nwy0 = jnp.where(acc, plsc.load_gather(bx, [widx + 1]), 0.0)
            nwx1 = jnp.where(acc, plsc.load_gather(bx, [widx + 2]), 0.0)
            nwy1 = jnp.where(acc, plsc.load_gather(bx, [widx + 3]), 0.0)
            nwa = (jnp.maximum(nwx1 - nwx0, 0.0)
                   * jnp.maximum(nwy1 - nwy0, 0.0))
            new_wins += [nwx0, nwy0, nwx1, nwy1, nwa]

            @pl.when(jnp.logical_and(acc, is_out))
            def _():
                row = jnp.where(iota == 0, nwx0,
                      jnp.where(iota == 1, nwy0,
                      jnp.where(iota == 2, nwx1,
                      jnp.where(iota == 3, nwy1,
                      jnp.where(iota == 4, jnp.broadcast_to(m_t, (_L,)),
                                0.0)))))
                plsc.store_scatter(outbuf, [(kc + ka) * 5 + iota], row,
                                   mask=iota < 5)

            psup = _sup_vs(nwx0, nwy0, nwx1, nwy1, nwa,
                           px0, py0, px1, py1, parea)
            pv = jnp.where(jnp.logical_and(acc, psup), _NEG, pv)
            ka = ka + acc.astype(jnp.int32)

        return tuple([rnd + 1, kc + ka, ka] + new_wins)

    init = tuple([jnp.int32(0), jnp.int32(0), jnp.int32(1)]
                 + [jnp.zeros((_L,), jnp.float32)] * (5 * _B))
    final = lax.while_loop(cond_fn, round_fn, init)
    kc_end = final[1]

    @pl.when(is_out)
    def _():
        def fill(r, c):
            plsc.store_scatter(outbuf, [r * 5 + iota], zero, mask=iota < 5)
            return c
        lax.fori_loop(kc_end, _K, fill, 0)
        pltpu.sync_copy(outbuf.at[pl.ds(0, 5 * _K)], out_hbm)


@jax.jit
def kernel(boxes, scores):
    flat = boxes.reshape(4 * _N)
    mesh = plsc.VectorSubcoreMesh(core_axis_name="c", subcore_axis_name="s",
                                  num_cores=2, num_subcores=_NSUB)
    f = pl.kernel(
        _sc_body,
        out_type=jax.ShapeDtypeStruct((5 * _K,), jnp.float32),
        mesh=mesh,
        compiler_params=pltpu.CompilerParams(needs_layout_passes=False),
        scratch_types=[
            pltpu.VMEM((4 * _NPAD,), jnp.float32),
            pltpu.VMEM((_NPAD,), jnp.float32),
            pltpu.VMEM((_PER,), jnp.float32),
            pltpu.VMEM((_PER,), jnp.float32),
            pltpu.VMEM((_PER,), jnp.float32),
            pltpu.VMEM((_PER,), jnp.float32),
            pltpu.VMEM((_PER,), jnp.float32),
            pltpu.VMEM((_L,), jnp.float32),
            pltpu.VMEM((_SLOTS,), jnp.float32),
            pltpu.VMEM((512,), jnp.float32),
            pltpu.VMEM_SHARED((2 * _SLOTS,), jnp.float32),
        ],
    )
    out = f(flat, scores)
    return out.reshape(_K, 5)
